# Initial kernel scaffold; baseline (speedup 1.0000x reference)
import jax
import jax.numpy as jnp
from jax.experimental import pallas as pl


def kernel(x, embedding, W_proj, b_proj, W1, a_src1, a_dst1, b1, W2, a_src2, a_dst2, b2):
    raise NotImplementedError("write your pallas kernel here")


# speedup vs baseline: 7999.8403x; 7999.8403x over previous
import functools
import jax
import jax.numpy as jnp
from jax import lax
from jax.experimental import pallas as pl
from jax.experimental.pallas import tpu as pltpu

RB = 256


def _adj_body(emb_full, emb_rows, adj_out, m1_out, m2_out, *, N):
    rb = pl.program_id(0)
    emb = emb_full[...]
    nf = emb * lax.rsqrt(jnp.maximum(jnp.sum(emb * emb, axis=1, keepdims=True), 1e-12))
    er = emb_rows[...]
    nr = er * lax.rsqrt(jnp.maximum(jnp.sum(er * er, axis=1, keepdims=True), 1e-12))
    adjb = lax.dot_general(nr, nf, (((1,), (1,)), ((), ())),
                           preferred_element_type=jnp.float32)
    col = lax.broadcasted_iota(jnp.int32, (RB, N), 1)
    grow = rb * RB + lax.broadcasted_iota(jnp.int32, (RB, N), 0)
    adjb = jnp.where(col == grow, 0.0, adjb)
    m1 = jnp.max(adjb, axis=1, keepdims=True)
    j0 = jnp.min(jnp.where(adjb == m1, col, N), axis=1, keepdims=True)
    m2 = jnp.max(jnp.where(col == j0, jnp.float32(-jnp.inf), adjb),
                 axis=1, keepdims=True)
    adj_out[...] = adjb
    m1_out[...] = m1
    m2_out[...] = m2


def _gat_body(x, Wp, bp, Wf, asrc, adst, bias, adj, m1v, m2v, out,
              hall_s, scal_s, *, N, H, O, proj, relu_out):
    rb = pl.program_id(1)

    @pl.when(rb == 0)
    def _():
        xb = x[0]
        if proj:
            xb = jnp.dot(xb, Wp[...], preferred_element_type=jnp.float32) + bp[...]
        hall_s[...] = jnp.dot(xb, Wf[...], preferred_element_type=jnp.float32)
        mv = jnp.min(m2v[...])
        m1a = m1v[...]
        anyr = jnp.any((m1a >= mv) & (m1a > 0.5))
        scal_s[0] = mv
        scal_s[1] = jnp.where(anyr, 1.0, 0.0)

    mv = scal_s[0]
    anyr = scal_s[1]
    adjb = adj[...]
    col = lax.broadcasted_iota(jnp.int32, (RB, N), 1)
    grow = rb * RB + lax.broadcasted_iota(jnp.int32, (RB, N), 0)
    realf = jnp.where((adjb >= mv) & (adjb > 0.5), 1.0, 0.0)
    eyef = jnp.where(col == grow, 1.0, 0.0)
    mask = jnp.where(anyr > 0.5, realf, eyef) > 0.5

    ones_rb = jnp.ones((RB, 1), jnp.float32)
    parts = []
    for h in range(H):
        hh_rows = hall_s[pl.ds(rb * RB, RB), h * O:(h + 1) * O]
        hh_all = hall_s[:, h * O:(h + 1) * O]
        dcol = jnp.dot(hh_all, adst[:, h:h + 1],
                       preferred_element_type=jnp.float32)
        A = jnp.concatenate([hh_rows, ones_rb], axis=1)
        Bm = jnp.concatenate(
            [jnp.broadcast_to(asrc[h:h + 1, :], (N, O)), dcol], axis=1)
        e0 = lax.dot_general(A, Bm, (((1,), (1,)), ((), ())),
                             preferred_element_type=jnp.float32)
        e = jnp.where(e0 >= 0, e0, 0.2 * e0)
        m = jnp.max(jnp.where(mask, e, jnp.float32(-jnp.inf)), axis=1, keepdims=True)
        m = jnp.where(m > -1e37, m, 0.0)
        p = jnp.where(mask, jnp.exp(e - m), 0.0)
        den = jnp.sum(p, axis=1, keepdims=True)
        oh = jnp.dot(p, hh_all, preferred_element_type=jnp.float32)
        oh = oh / (den + 1e-9) + bias[:, h * O:(h + 1) * O]
        parts.append(oh)
    res = jnp.concatenate(parts, axis=1)
    if relu_out:
        res = jnp.maximum(res, 0.0)
    out[...] = res[None]


def _gat_layer(xin, Wp, bp, Wflat, asrc_r, adst_c, bias, adj, m1v, m2v,
               *, proj, relu_out):
    B, N, F = xin.shape
    H = asrc_r.shape[0]
    O = asrc_r.shape[1]
    HO = H * O
    nblk = N // RB
    grid = (B, nblk)
    body = functools.partial(_gat_body, N=N, H=H, O=O, proj=proj, relu_out=relu_out)
    return pl.pallas_call(
        body,
        grid=grid,
        in_specs=[
            pl.BlockSpec((1, N, F), lambda b, rb: (b, 0, 0)),
            pl.BlockSpec(Wp.shape, lambda b, rb: (0, 0)),
            pl.BlockSpec(bp.shape, lambda b, rb: (0, 0)),
            pl.BlockSpec(Wflat.shape, lambda b, rb: (0, 0)),
            pl.BlockSpec(asrc_r.shape, lambda b, rb: (0, 0)),
            pl.BlockSpec(adst_c.shape, lambda b, rb: (0, 0)),
            pl.BlockSpec(bias.shape, lambda b, rb: (0, 0)),
            pl.BlockSpec((RB, N), lambda b, rb: (rb, 0)),
            pl.BlockSpec((N, 1), lambda b, rb: (0, 0)),
            pl.BlockSpec((N, 1), lambda b, rb: (0, 0)),
        ],
        out_specs=pl.BlockSpec((1, RB, HO), lambda b, rb: (b, rb, 0)),
        out_shape=jax.ShapeDtypeStruct((B, N, HO), jnp.float32),
        scratch_shapes=[
            pltpu.VMEM((N, HO), jnp.float32),
            pltpu.SMEM((2,), jnp.float32),
        ],
        compiler_params=pltpu.CompilerParams(
            dimension_semantics=("arbitrary", "arbitrary")),
    )(xin, Wp, bp, Wflat, asrc_r, adst_c, bias, adj, m1v, m2v)


def kernel(x, embedding, W_proj, b_proj, W1, a_src1, a_dst1, b1,
           W2, a_src2, a_dst2, b2):
    B, N, IN = x.shape
    HID = embedding.shape[1]
    H = W1.shape[0]
    nblk = N // RB

    adj, m1v, m2v = pl.pallas_call(
        functools.partial(_adj_body, N=N),
        grid=(nblk,),
        in_specs=[
            pl.BlockSpec((N, HID), lambda rb: (0, 0)),
            pl.BlockSpec((RB, HID), lambda rb: (rb, 0)),
        ],
        out_specs=[
            pl.BlockSpec((RB, N), lambda rb: (rb, 0)),
            pl.BlockSpec((RB, 1), lambda rb: (rb, 0)),
            pl.BlockSpec((RB, 1), lambda rb: (rb, 0)),
        ],
        out_shape=[
            jax.ShapeDtypeStruct((N, N), jnp.float32),
            jax.ShapeDtypeStruct((N, 1), jnp.float32),
            jax.ShapeDtypeStruct((N, 1), jnp.float32),
        ],
        compiler_params=pltpu.CompilerParams(
            dimension_semantics=("arbitrary",)),
    )(embedding, embedding)

    Wflat1 = jnp.transpose(W1, (1, 0, 2)).reshape(HID, HID)
    Wflat2 = jnp.transpose(W2, (1, 0, 2)).reshape(HID, -1)
    bp = b_proj.reshape(1, -1)
    bias1 = b1.reshape(1, -1)
    bias2 = b2.reshape(1, -1)

    h1 = _gat_layer(x, W_proj, bp, Wflat1, a_src1, a_dst1.T, bias1,
                    adj, m1v, m2v, proj=True, relu_out=True)
    out = _gat_layer(h1, W_proj, bp, Wflat2, a_src2, a_dst2.T, bias2,
                     adj, m1v, m2v, proj=False, relu_out=False)
    return out
